# baseline (speedup 1.0000x reference)
import functools

import numpy as np

import jax
import jax.numpy as jnp
from jax import lax
from jax.experimental import pallas as pl
from jax.experimental.pallas import tpu as pltpu
from jax.experimental.pallas import tpu_sc as plsc

N, E, D = 10000, 320000, 128
NC, NS = 2, 16
NW = NC * NS
CHUNK = 80
CHUNKS = 128
EPW = CHUNK * CHUNKS
E_PAD = EPW * NW
N_PAD = 10112
RPT = N_PAD // NS
_HIGH = lax.Precision.HIGHEST
_MESH = plsc.VectorSubcoreMesh(core_axis_name="c", subcore_axis_name="s")
_SLICES = tuple([CHUNK] * (RPT // CHUNK) +
                ([RPT % CHUNK] if RPT % CHUNK else []))


def _acc_slice_copy(src_at, dst_at, base):
  off = 0
  for sz in _SLICES:
    pltpu.sync_copy(src_at(base + off, sz), dst_at(base + off, sz))
    off += sz


@functools.partial(
    pl.kernel,
    out_type=(jax.ShapeDtypeStruct((NC, N_PAD, D), jnp.float32),),
    mesh=_MESH,
    scratch_types=[
        pltpu.VMEM((8, CHUNK), jnp.int32),
        pltpu.VMEM((8, CHUNK), jnp.int32),
        pltpu.VMEM((CHUNK, D), jnp.float32),
        pltpu.VMEM((CHUNK, D), jnp.float32),
        pltpu.VMEM_SHARED((N_PAD, D), jnp.float32),
        pltpu.SemaphoreType.DMA,
        pltpu.SemaphoreType.DMA,
    ])
def _sc_agg(xp_hbm, src_hbm, dst_hbm, zeros_hbm, acc_out,
            src_v, dst_v, rowbuf, rowbuf2, acc_sh, sem, sem2):
  c = lax.axis_index("c")
  s = lax.axis_index("s")
  wid = s * NC + c
  base = s * RPT
  rb = (rowbuf, rowbuf2)

  _acc_slice_copy(lambda r, sz: zeros_hbm.at[pl.ds(0, sz)],
                  lambda r, sz: acc_sh.at[pl.ds(r, sz)], base)
  plsc.subcore_barrier()

  def chunk_step(t, carry):
    pltpu.sync_copy(src_hbm.at[wid, pl.ds(t * 8, 8)], src_v)
    pltpu.sync_copy(dst_hbm.at[wid, pl.ds(t * 8, 8)], dst_v)
    pltpu.async_copy(xp_hbm.at[src_v.at[0]], rb[0], sem)
    for k in range(8):
      if k + 1 < 8:
        if k >= 1:
          pltpu.make_async_copy(rb[(k - 1) % 2], acc_sh.at[dst_v.at[k - 1]],
                                sem2).wait()
        pltpu.async_copy(xp_hbm.at[src_v.at[k + 1]], rb[(k + 1) % 2], sem)
      pltpu.make_async_copy(xp_hbm.at[src_v.at[k]], rb[k % 2], sem).wait()
      pltpu.async_copy(rb[k % 2], acc_sh.at[dst_v.at[k]], sem2, add=True)
    pltpu.make_async_copy(rb[0], acc_sh.at[dst_v.at[6]], sem2).wait()
    pltpu.make_async_copy(rb[1], acc_sh.at[dst_v.at[7]], sem2).wait()
    return carry
  lax.fori_loop(0, CHUNKS // 8, chunk_step, 0)

  plsc.subcore_barrier()

  _acc_slice_copy(lambda r, sz: acc_sh.at[pl.ds(r, sz)],
                  lambda r, sz: acc_out.at[c, pl.ds(r, sz)], base)


CHUNK_C = 128
CHUNKS_C = EPW // CHUNK_C


@functools.partial(
    pl.kernel,
    out_type=(jax.ShapeDtypeStruct((NC, N_PAD, D), jnp.float32),),
    mesh=_MESH,
    scratch_types=[
        pltpu.VMEM((8, CHUNK_C), jnp.int32),
        pltpu.VMEM((CHUNK_C, D), jnp.float32),
        pltpu.VMEM_SHARED((N_PAD, D), jnp.float32),
    ])
def _sc_cnt(dst_hbm, zeros_hbm, ones_hbm, cnt_out,
            dst_v, onesrows, acc_sh):
  c = lax.axis_index("c")
  s = lax.axis_index("s")
  wid = s * NC + c
  base = s * RPT

  pltpu.sync_copy(ones_hbm, onesrows)
  _acc_slice_copy(lambda r, sz: zeros_hbm.at[pl.ds(0, sz)],
                  lambda r, sz: acc_sh.at[pl.ds(r, sz)], base)
  plsc.subcore_barrier()

  def chunk_step(t, carry):
    pltpu.sync_copy(dst_hbm.at[wid, pl.ds(t * 8, 8)], dst_v)
    for k in range(8):
      pltpu.sync_copy(onesrows, acc_sh.at[dst_v.at[k]], add=True)
    return carry
  lax.fori_loop(0, CHUNKS_C // 8, chunk_step, 0)

  plsc.subcore_barrier()

  _acc_slice_copy(lambda r, sz: acc_sh.at[pl.ds(r, sz)],
                  lambda r, sz: cnt_out.at[c, pl.ds(r, sz)], base)


_BLK = 2528


def _dot(a, b):
  return jnp.dot(a, b, preferred_element_type=jnp.float32)


def _proj_body(x_ref, w_ref, b_ref, o_ref):
  o_ref[...] = jnp.maximum(_dot(x_ref[...], w_ref[...]) + b_ref[...], 0.0)


def _proj(x, W, b):
  return pl.pallas_call(
      _proj_body,
      grid=(N_PAD // _BLK,),
      in_specs=[
          pl.BlockSpec((_BLK, D), lambda i: (i, 0)),
          pl.BlockSpec((D, D), lambda i: (0, 0)),
          pl.BlockSpec((D,), lambda i: (0,)),
      ],
      out_specs=pl.BlockSpec((_BLK, D), lambda i: (i, 0)),
      out_shape=jax.ShapeDtypeStruct((N_PAD, D), jnp.float32),
  )(x, W, b)


def _mean_agg(acc_ref, cnt_ref):
  ssum = acc_ref[0] + acc_ref[1]
  cnt = cnt_ref[0][:, 0:1] + cnt_ref[1][:, 0:1]
  return ssum / jnp.clip(cnt, 1.0, None)


def _combine1_body(acc_ref, cnt_ref, x_ref, wl_ref, bl_ref, wr_ref,
                   wp2_ref, bp2_ref, h_ref, xp2_ref):
  agg = _mean_agg(acc_ref, cnt_ref)
  h = jnp.maximum(
      _dot(agg, wl_ref[...]) + bl_ref[...] + _dot(x_ref[...], wr_ref[...]),
      0.0)
  h_ref[...] = h
  xp2_ref[...] = jnp.maximum(_dot(h, wp2_ref[...]) + bp2_ref[...], 0.0)


def _combine1(acc, cnt, x, Wl, bl, Wr, Wp2, bp2):
  return pl.pallas_call(
      _combine1_body,
      grid=(N_PAD // _BLK,),
      in_specs=[
          pl.BlockSpec((NC, _BLK, D), lambda i: (0, i, 0)),
          pl.BlockSpec((NC, _BLK, D), lambda i: (0, i, 0)),
          pl.BlockSpec((_BLK, D), lambda i: (i, 0)),
          pl.BlockSpec((D, D), lambda i: (0, 0)),
          pl.BlockSpec((D,), lambda i: (0,)),
          pl.BlockSpec((D, D), lambda i: (0, 0)),
          pl.BlockSpec((D, D), lambda i: (0, 0)),
          pl.BlockSpec((D,), lambda i: (0,)),
      ],
      out_specs=[
          pl.BlockSpec((_BLK, D), lambda i: (i, 0)),
          pl.BlockSpec((_BLK, D), lambda i: (i, 0)),
      ],
      out_shape=[
          jax.ShapeDtypeStruct((N_PAD, D), jnp.float32),
          jax.ShapeDtypeStruct((N_PAD, D), jnp.float32),
      ],
  )(acc, cnt, x, Wl, bl, Wr, Wp2, bp2)


def _combine2_body(acc_ref, cnt_ref, h_ref, wl_ref, bl_ref, wr_ref, o_ref):
  agg = _mean_agg(acc_ref, cnt_ref)
  o = (_dot(agg, wl_ref[...]) + bl_ref[...] + _dot(h_ref[...], wr_ref[...]))
  norm = jnp.sqrt(jnp.sum(o * o, axis=-1, keepdims=True))
  o = o / jnp.clip(norm, 1e-12, None)
  o = jnp.maximum(o, 0.0)
  m = jnp.max(o, axis=-1, keepdims=True)
  lse = m + jnp.log(jnp.sum(jnp.exp(o - m), axis=-1, keepdims=True))
  o_ref[...] = o - lse


def _combine2(acc, cnt, h, Wl, bl, Wr):
  return pl.pallas_call(
      _combine2_body,
      grid=(N_PAD // _BLK,),
      in_specs=[
          pl.BlockSpec((NC, _BLK, D), lambda i: (0, i, 0)),
          pl.BlockSpec((NC, _BLK, D), lambda i: (0, i, 0)),
          pl.BlockSpec((_BLK, D), lambda i: (i, 0)),
          pl.BlockSpec((D, D), lambda i: (0, 0)),
          pl.BlockSpec((D,), lambda i: (0,)),
          pl.BlockSpec((D, D), lambda i: (0, 0)),
      ],
      out_specs=pl.BlockSpec((_BLK, D), lambda i: (i, 0)),
      out_shape=jax.ShapeDtypeStruct((N_PAD, D), jnp.float32),
  )(acc, cnt, h, Wl, bl, Wr)


def kernel(matrix_nodes_features, edge_index, Wp1, bp1, Wl1, bl1, Wr1,
           Wp2, bp2, Wl2, bl2, Wr2):
  x = jnp.pad(matrix_nodes_features, ((0, N_PAD - N), (0, 0)))
  pad_i = np.arange(E_PAD - E, dtype=np.int32)
  pad_src = jnp.asarray((pad_i * 131) % N)
  pad_dst = jnp.asarray(N + pad_i % (N_PAD - N))
  dst_flat = jnp.concatenate([edge_index[1], pad_dst])
  src = jnp.concatenate([edge_index[0], pad_src]).reshape(NW, CHUNKS, CHUNK)
  dst = dst_flat.reshape(NW, CHUNKS, CHUNK)
  dst_c = dst_flat.reshape(NW, CHUNKS_C, CHUNK_C)
  zeros = jnp.zeros((CHUNK, D), jnp.float32)
  ones = jnp.ones((CHUNK_C, D), jnp.float32)

  (cnt,) = _sc_cnt(dst_c, zeros, ones)
  xp1 = _proj(x, Wp1, bp1)
  (acc1,) = _sc_agg(xp1, src, dst, zeros)
  h, xp2 = _combine1(acc1, cnt, x, Wl1, bl1, Wr1, Wp2, bp2)
  (acc2,) = _sc_agg(xp2, src, dst, zeros)
  out = _combine2(acc2, cnt, h, Wl2, bl2, Wr2)
  return out[:N]
